# baseline (speedup 1.0000x reference)
import functools

import jax
import jax.numpy as jnp
from jax import lax
from jax.experimental import pallas as pl
from jax.experimental.pallas import tpu as pltpu
from jax.experimental.pallas import tpu_sc as plsc

N = 100000
NBINS = 512
NW = 16
CHUNK = 6272
NVEC = CHUNK // 16
OVERLAP = NW * CHUNK - N

_mesh = plsc.VectorSubcoreMesh(core_axis_name="c", subcore_axis_name="s", num_cores=1)


@functools.partial(
    pl.kernel,
    out_type=(
        jax.ShapeDtypeStruct((64, 128), jnp.float32),
        jax.ShapeDtypeStruct((NW, 128), jnp.float32),
    ),
    mesh=_mesh,
    scratch_types=[
        pltpu.VMEM((CHUNK,), jnp.float32),
        pltpu.VMEM((CHUNK,), jnp.int32),
        pltpu.VMEM((16 * NBINS,), jnp.float32),
        pltpu.VMEM((NBINS,), jnp.float32),
        pltpu.VMEM((128,), jnp.float32),
        pltpu.SemaphoreType.DMA,
        pltpu.SemaphoreType.DMA,
    ],
    compiler_params=pltpu.CompilerParams(
        use_tc_tiling_on_sc=False, needs_layout_passes=False
    ),
)
def _sc_partials(beta_hbm, pid_hbm, seg_out, noise_out, beta_v, pid_v, acc,
                 res, nres, sem0, sem1):
    wid = lax.axis_index("s") + lax.axis_index("c")
    last = wid == NW - 1
    base = jnp.where(last, N - CHUNK, wid * CHUNK)
    skip = jnp.where(last, OVERLAP, 0)

    cp0 = pltpu.async_copy(beta_hbm.at[pl.ds(base, CHUNK)], beta_v, sem0)
    cp1 = pltpu.async_copy(pid_hbm.at[pl.ds(base, CHUNK)], pid_v, sem1)

    neg = jnp.full((16,), -1.0, jnp.float32)

    def init_body(i, _):
        b = i * 256
        for j in range(16):
            acc[pl.ds(b + j * 16, 16)] = neg
        return 0

    lax.fori_loop(0, 16 * NBINS // 256, init_body, 0)
    cp0.wait()
    cp1.wait()

    loff = lax.iota(jnp.int32, 16) * NBINS
    zf = jnp.zeros((16,), jnp.float32)

    def body(i, carry):
        ns, nc = carry
        for u in range(4):
            off = (4 * i + u) * 16
            pv = pid_v[pl.ds(off, 16)]
            bv = beta_v[pl.ds(off, 16)]
            idx = loff + pv
            cur = plsc.load_gather(acc, [idx])
            plsc.store_scatter(acc, [idx], jnp.maximum(cur, bv))
            m = (pv == 0) & (off >= skip)
            ns = ns + jnp.where(m, bv, 0.0)
            nc = nc + jnp.where(m, 1.0, 0.0)
        return ns, nc

    ns, nc = lax.fori_loop(0, NVEC // 4, body, (zf, zf))

    def red_body(cb, _):
        b = cb * 16
        m = acc[pl.ds(b, 16)]
        for l in range(1, 16):
            m = jnp.maximum(m, acc[pl.ds(l * NBINS + b, 16)])
        res[pl.ds(b, 16)] = m
        return 0

    lax.fori_loop(0, NBINS // 16, red_body, 0)
    for j in range(4):
        pltpu.sync_copy(res.at[pl.ds(j * 128, 128)], seg_out.at[4 * wid + j])

    nres[pl.ds(0, 16)] = ns
    nres[pl.ds(16, 16)] = nc
    for i in range(2, 8):
        nres[pl.ds(i * 16, 16)] = zf
    pltpu.sync_copy(nres, noise_out.at[wid])


def _combine_body(seg_ref, noise_ref, o_ref):
    x = seg_ref[:, :]
    row = lax.broadcasted_iota(jnp.int32, (64, 128), 0)
    ck = lax.broadcasted_iota(jnp.int32, (1, 128), 1)
    bsum = jnp.float32(0.0)
    npres = jnp.float32(0.0)
    for k in range(4):
        mk = jnp.max(jnp.where(row % 4 == k, x, -1.0), axis=0, keepdims=True)
        pres = (mk >= 0.0) & (ck + 128 * k >= 1)
        bsum += jnp.sum(jnp.where(pres, 1.0 - mk, 0.0))
        npres += jnp.sum(pres.astype(jnp.float32))
    nz = noise_ref[:, :]
    ncol = lax.broadcasted_iota(jnp.int32, (NW, 128), 1)
    ns = jnp.sum(jnp.where(ncol < 16, nz, 0.0))
    nc = jnp.sum(jnp.where((ncol >= 16) & (ncol < 32), nz, 0.0))
    loss = bsum / npres
    noise_mean = ns / jnp.maximum(nc, 1.0)
    loss = jnp.where(nc > 0.5, loss + 0.1 * noise_mean, loss)
    o_ref[0, 0] = loss


_combine = pl.pallas_call(
    _combine_body,
    out_shape=jax.ShapeDtypeStruct((1, 1), jnp.float32),
    out_specs=pl.BlockSpec(memory_space=pltpu.SMEM),
)


def kernel(beta, particle_id):
    seg, noise = _sc_partials(beta, particle_id)
    return _combine(seg, noise)[0, 0]


# speedup vs baseline: 1.0218x; 1.0218x over previous
import functools

import jax
import jax.numpy as jnp
from jax import lax
from jax.experimental import pallas as pl
from jax.experimental.pallas import tpu as pltpu
from jax.experimental.pallas import tpu_sc as plsc

N = 100000
NBINS = 512
NW = 32
CHUNK = 3136
NVEC = CHUNK // 16
OVERLAP = NW * CHUNK - N

_mesh = plsc.VectorSubcoreMesh(core_axis_name="c", subcore_axis_name="s")


@functools.partial(
    pl.kernel,
    out_type=(
        jax.ShapeDtypeStruct((128, 128), jnp.float32),
        jax.ShapeDtypeStruct((NW, 128), jnp.float32),
    ),
    mesh=_mesh,
    scratch_types=[
        pltpu.VMEM((CHUNK,), jnp.float32),
        pltpu.VMEM((CHUNK,), jnp.int32),
        pltpu.VMEM((16 * NBINS,), jnp.float32),
        pltpu.VMEM((16 * NBINS,), jnp.float32),
        pltpu.VMEM((NBINS,), jnp.float32),
        pltpu.VMEM((128,), jnp.float32),
        pltpu.SemaphoreType.DMA,
        pltpu.SemaphoreType.DMA,
        pltpu.SemaphoreType.DMA,
    ],
    compiler_params=pltpu.CompilerParams(
        use_tc_tiling_on_sc=False, needs_layout_passes=False
    ),
)
def _sc_partials(beta_hbm, pid_hbm, seg_out, noise_out, beta_v, pid_v, acc,
                 acc2, res, nres, sem0, sem1, sem2):
    wid = lax.axis_index("s") * 2 + lax.axis_index("c")
    last = wid == NW - 1
    base = jnp.where(last, N - CHUNK, wid * CHUNK)
    skip = jnp.where(last, OVERLAP, 0)

    cp0 = pltpu.async_copy(beta_hbm.at[pl.ds(base, CHUNK)], beta_v, sem0)
    cp1 = pltpu.async_copy(pid_hbm.at[pl.ds(base, CHUNK)], pid_v, sem1)

    neg = jnp.full((16,), -1.0, jnp.float32)

    def init_body(i, _):
        b = i * 256
        for j in range(16):
            acc[pl.ds(b + j * 16, 16)] = neg
            acc2[pl.ds(b + j * 16, 16)] = neg
        return 0

    lax.fori_loop(0, 16 * NBINS // 256, init_body, 0)
    cp0.wait()
    cp1.wait()

    loff = lax.iota(jnp.int32, 16) * NBINS
    zf = jnp.zeros((16,), jnp.float32)

    def body(i, carry):
        ns, nc = carry
        for u in range(4):
            off = (4 * i + u) * 16
            a = acc if u % 2 == 0 else acc2
            pv = pid_v[pl.ds(off, 16)]
            bv = beta_v[pl.ds(off, 16)]
            idx = loff + pv
            cur = plsc.load_gather(a, [idx])
            plsc.store_scatter(a, [idx], jnp.maximum(cur, bv))
            m = (pv == 0) & (off >= skip)
            ns = ns + jnp.where(m, bv, 0.0)
            nc = nc + jnp.where(m, 1.0, 0.0)
        return ns, nc

    ns, nc = lax.fori_loop(0, NVEC // 4, body, (zf, zf))

    def red_body(cb, _):
        b = cb * 16
        m = jnp.maximum(acc[pl.ds(b, 16)], acc2[pl.ds(b, 16)])
        for l in range(1, 16):
            m = jnp.maximum(m, acc[pl.ds(l * NBINS + b, 16)])
            m = jnp.maximum(m, acc2[pl.ds(l * NBINS + b, 16)])
        res[pl.ds(b, 16)] = m
        return 0

    lax.fori_loop(0, NBINS // 16, red_body, 0)

    nres[pl.ds(0, 16)] = ns
    nres[pl.ds(16, 16)] = nc
    for i in range(2, 8):
        nres[pl.ds(i * 16, 16)] = zf

    cps = [pltpu.async_copy(res.at[pl.ds(j * 128, 128)],
                            seg_out.at[4 * wid + j], sem2)
           for j in range(4)]
    cps.append(pltpu.async_copy(nres, noise_out.at[wid], sem2))
    for cp in cps:
        cp.wait()


def _combine_body(seg_ref, noise_ref, o_ref):
    x = seg_ref[:, :]
    row = lax.broadcasted_iota(jnp.int32, (128, 128), 0)
    ck = lax.broadcasted_iota(jnp.int32, (1, 128), 1)
    bsum = jnp.float32(0.0)
    npres = jnp.float32(0.0)
    for k in range(4):
        mk = jnp.max(jnp.where(row % 4 == k, x, -1.0), axis=0, keepdims=True)
        pres = (mk >= 0.0) & (ck + 128 * k >= 1)
        bsum += jnp.sum(jnp.where(pres, 1.0 - mk, 0.0))
        npres += jnp.sum(pres.astype(jnp.float32))
    nz = noise_ref[:, :]
    ncol = lax.broadcasted_iota(jnp.int32, (NW, 128), 1)
    ns = jnp.sum(jnp.where(ncol < 16, nz, 0.0))
    nc = jnp.sum(jnp.where((ncol >= 16) & (ncol < 32), nz, 0.0))
    loss = bsum / npres
    noise_mean = ns / jnp.maximum(nc, 1.0)
    loss = jnp.where(nc > 0.5, loss + 0.1 * noise_mean, loss)
    o_ref[0, 0] = loss


_combine = pl.pallas_call(
    _combine_body,
    out_shape=jax.ShapeDtypeStruct((1, 1), jnp.float32),
    out_specs=pl.BlockSpec(memory_space=pltpu.SMEM),
)


def kernel(beta, particle_id):
    seg, noise = _sc_partials(beta, particle_id)
    return _combine(seg, noise)[0, 0]


# speedup vs baseline: 1.0321x; 1.0101x over previous
import functools

import jax
import jax.numpy as jnp
from jax import lax
from jax.experimental import pallas as pl
from jax.experimental.pallas import tpu as pltpu
from jax.experimental.pallas import tpu_sc as plsc

N = 100000
NBINS = 512
NW = 32
CHUNK = 3136
NVEC = CHUNK // 16
OVERLAP = NW * CHUNK - N

_mesh = plsc.VectorSubcoreMesh(core_axis_name="c", subcore_axis_name="s")


@functools.partial(
    pl.kernel,
    out_type=(
        jax.ShapeDtypeStruct((128, 128), jnp.float32),
        jax.ShapeDtypeStruct((NW, 128), jnp.float32),
    ),
    mesh=_mesh,
    scratch_types=[
        pltpu.VMEM((CHUNK,), jnp.float32),
        pltpu.VMEM((CHUNK,), jnp.int32),
        pltpu.VMEM((16 * NBINS,), jnp.float32),
        pltpu.VMEM((NBINS,), jnp.float32),
        pltpu.VMEM((128,), jnp.float32),
        pltpu.SemaphoreType.DMA,
        pltpu.SemaphoreType.DMA,
        pltpu.SemaphoreType.DMA,
    ],
    compiler_params=pltpu.CompilerParams(
        use_tc_tiling_on_sc=False, needs_layout_passes=False
    ),
)
def _sc_partials(beta_hbm, pid_hbm, seg_out, noise_out, beta_v, pid_v, acc,
                 res, nres, sem0, sem1, sem2):
    wid = lax.axis_index("s") * 2 + lax.axis_index("c")
    last = wid == NW - 1
    base = jnp.where(last, N - CHUNK, wid * CHUNK)
    skip = jnp.where(last, OVERLAP, 0)

    cp0 = pltpu.async_copy(beta_hbm.at[pl.ds(base, CHUNK)], beta_v, sem0)
    cp1 = pltpu.async_copy(pid_hbm.at[pl.ds(base, CHUNK)], pid_v, sem1)

    neg = jnp.full((16,), -1.0, jnp.float32)

    def init_body(i, _):
        b = i * 256
        for j in range(16):
            acc[pl.ds(b + j * 16, 16)] = neg
        return 0

    lax.fori_loop(0, 16 * NBINS // 256, init_body, 0)
    cp0.wait()
    cp1.wait()

    loff = lax.iota(jnp.int32, 16) * NBINS
    zf = jnp.zeros((16,), jnp.float32)

    def body(i, carry):
        ns, nc = carry
        for u in range(7):
            off = (7 * i + u) * 16
            pv = pid_v[pl.ds(off, 16)]
            bv = beta_v[pl.ds(off, 16)]
            idx = loff + pv
            cur = plsc.load_gather(acc, [idx])
            plsc.store_scatter(acc, [idx], jnp.maximum(cur, bv))
            m = (pv == 0) & (off >= skip)
            ns = ns + jnp.where(m, bv, 0.0)
            nc = nc + jnp.where(m, 1.0, 0.0)
        return ns, nc

    ns, nc = lax.fori_loop(0, NVEC // 7, body, (zf, zf))

    def red_body(cb, _):
        for q in range(2):
            b = (2 * cb + q) * 16
            m = acc[pl.ds(b, 16)]
            for l in range(1, 16):
                m = jnp.maximum(m, acc[pl.ds(l * NBINS + b, 16)])
            res[pl.ds(b, 16)] = m
        return 0

    lax.fori_loop(0, NBINS // 32, red_body, 0)

    nres[pl.ds(0, 16)] = ns
    nres[pl.ds(16, 16)] = nc
    for i in range(2, 8):
        nres[pl.ds(i * 16, 16)] = zf

    cps = [pltpu.async_copy(res.at[pl.ds(j * 128, 128)],
                            seg_out.at[4 * wid + j], sem2)
           for j in range(4)]
    cps.append(pltpu.async_copy(nres, noise_out.at[wid], sem2))
    for cp in cps:
        cp.wait()


def _combine_body(seg_ref, noise_ref, o_ref):
    x = seg_ref[:, :]
    row = lax.broadcasted_iota(jnp.int32, (128, 128), 0)
    ck = lax.broadcasted_iota(jnp.int32, (1, 128), 1)
    bsum = jnp.float32(0.0)
    npres = jnp.float32(0.0)
    for k in range(4):
        mk = jnp.max(jnp.where(row % 4 == k, x, -1.0), axis=0, keepdims=True)
        pres = (mk >= 0.0) & (ck + 128 * k >= 1)
        bsum += jnp.sum(jnp.where(pres, 1.0 - mk, 0.0))
        npres += jnp.sum(pres.astype(jnp.float32))
    nz = noise_ref[:, :]
    ncol = lax.broadcasted_iota(jnp.int32, (NW, 128), 1)
    ns = jnp.sum(jnp.where(ncol < 16, nz, 0.0))
    nc = jnp.sum(jnp.where((ncol >= 16) & (ncol < 32), nz, 0.0))
    loss = bsum / npres
    noise_mean = ns / jnp.maximum(nc, 1.0)
    loss = jnp.where(nc > 0.5, loss + 0.1 * noise_mean, loss)
    o_ref[0, 0] = loss


_combine = pl.pallas_call(
    _combine_body,
    out_shape=jax.ShapeDtypeStruct((1, 1), jnp.float32),
    out_specs=pl.BlockSpec(memory_space=pltpu.SMEM),
)


def kernel(beta, particle_id):
    seg, noise = _sc_partials(beta, particle_id)
    return _combine(seg, noise)[0, 0]
